# baseline (speedup 1.0000x reference)
import functools

import jax
import jax.numpy as jnp
from jax import lax
from jax.experimental import pallas as pl
from jax.experimental.pallas import tpu as pltpu
from jax.experimental.pallas import tpu_sc as plsc

QP_NUM = 64
CHANNEL = 192
B, H, W = 32, 64, 64
HW = H * W
_XW = CHANNEL * 16
_PERW = CHANNEL * HW
_CH = 4 * HW
_NCHK = _PERW // _CH
_UNROLL = 8

_GWORKERS = 4
_GPER = B // _GWORKERS


def _prep_body(h_ref, a_ref, s_ref, t_ref):
  s_ref[...] = jax.nn.softplus(h_ref[...])
  t_ref[...] = jnp.tanh(a_ref[...])


def _tc_prep(h2, a2):
  return pl.pallas_call(
      _prep_body,
      out_shape=[jax.ShapeDtypeStruct((QP_NUM, CHANNEL), jnp.float32)] * 2,
  )(h2, a2)


def _sc_gather(s_exp, b_exp, t_exp, idx):
  mesh = plsc.VectorSubcoreMesh(core_axis_name="c", subcore_axis_name="s")

  @functools.partial(
      pl.kernel,
      mesh=mesh,
      out_type=[jax.ShapeDtypeStruct((B, _XW), jnp.float32)] * 3,
      scratch_types=[
          pltpu.VMEM((_GPER,), jnp.int32),
          pltpu.VMEM((_GPER, _XW), jnp.float32),
          pltpu.VMEM((_GPER, _XW), jnp.float32),
          pltpu.VMEM((_GPER, _XW), jnp.float32),
          pltpu.SemaphoreType.DMA,
      ],
  )
  def k(s_hbm, b_hbm, t_hbm, idx_hbm, os, ob, ot, idx_v, rs, rb, rt, sem):
    wid = lax.axis_index("s") * 2 + lax.axis_index("c")

    @pl.when(wid < _GWORKERS)
    def _():
      base = wid * _GPER
      pltpu.sync_copy(idx_hbm.at[pl.ds(base, _GPER)], idx_v)
      pltpu.async_copy(s_hbm.at[idx_v], rs, sem).wait()
      pltpu.async_copy(b_hbm.at[idx_v], rb, sem).wait()
      pltpu.async_copy(t_hbm.at[idx_v], rt, sem).wait()
      pltpu.sync_copy(rs, os.at[pl.ds(base, _GPER)])
      pltpu.sync_copy(rb, ob.at[pl.ds(base, _GPER)])
      pltpu.sync_copy(rt, ot.at[pl.ds(base, _GPER)])

  return k(s_exp, b_exp, t_exp, idx)


_HF = HW // 128
_CROWS = 4
_NCHK2 = (CHANNEL // 2) // _CROWS


def _sc_elementwise(x4, s_sel, b_sel, t_sel, nb, boff):
  mesh = plsc.VectorSubcoreMesh(core_axis_name="c", subcore_axis_name="s")
  hc = CHANNEL // 2

  @functools.partial(
      pl.kernel,
      mesh=mesh,
      out_type=jax.ShapeDtypeStruct((nb, CHANNEL, _HF, 128), jnp.float32),
      scratch_types=[
          pltpu.VMEM((_XW,), jnp.float32),
          pltpu.VMEM((_XW,), jnp.float32),
          pltpu.VMEM((_XW,), jnp.float32),
          pltpu.VMEM((2, _CROWS, _HF, 128), jnp.float32),
          pltpu.VMEM((2, _CROWS, _HF, 128), jnp.float32),
          pltpu.SemaphoreType.DMA,
          pltpu.SemaphoreType.DMA,
          pltpu.SemaphoreType.DMA,
          pltpu.SemaphoreType.DMA,
      ],
  )
  def k(x_hbm, s_hbm, b_hbm, t_hbm, o_hbm,
        s_buf, b_buf, t_buf, xbuf, obuf,
        in_sem0, in_sem1, out_sem0, out_sem1):
    in_sems = (in_sem0, in_sem1)
    out_sems = (out_sem0, out_sem1)
    wid = lax.axis_index("s") * 2 + lax.axis_index("c")
    bloc = wid // 2
    half = lax.rem(wid, 2)
    c_base = half * hc

    pltpu.sync_copy(s_hbm.at[boff + bloc], s_buf)
    pltpu.sync_copy(b_hbm.at[boff + bloc], b_buf)
    pltpu.sync_copy(t_hbm.at[boff + bloc], t_buf)

    def in_slice(chunk):
      return x_hbm.at[bloc, pl.ds(c_base + chunk * _CROWS, _CROWS)]

    def out_slice(chunk):
      return o_hbm.at[bloc, pl.ds(c_base + chunk * _CROWS, _CROWS)]

    pltpu.make_async_copy(in_slice(0), xbuf.at[0], in_sems[0]).start()
    pltpu.make_async_copy(in_slice(1), xbuf.at[1], in_sems[1]).start()

    def pair(g, carry):
      for buf in (0, 1):
        chunk = 2 * g + buf

        @pl.when(chunk >= 2)
        def _(buf=buf, chunk=chunk):
          pltpu.make_async_copy(
              obuf.at[buf], out_slice(chunk - 2), out_sems[buf]).wait()

        pltpu.make_async_copy(
            in_slice(chunk), xbuf.at[buf], in_sems[buf]).wait()

        xb = xbuf.at[buf]
        ob = obuf.at[buf]
        for r in range(_CROWS):
          c = c_base + chunk * _CROWS + r
          sv = s_buf[pl.ds(c * 16, 16)]
          bv = b_buf[pl.ds(c * 16, 16)]
          tv = t_buf[pl.ds(c * 16, 16)]
          t2v = tv + tv

          def inner(i, _, r=r, sv=sv, bv=bv, tv=tv, t2v=t2v, xb=xb, ob=ob):
            for u in range(128 // 16):
              v = xb[r, i, pl.ds(u * 16, 16)]
              y = v * sv + bv
              e = jnp.exp(y + y)
              q = t2v / (e + 1.0)
              ob[r, i, pl.ds(u * 16, 16)] = y + tv - q
            return 0

          lax.fori_loop(0, _HF, inner, 0)

        pltpu.make_async_copy(
            obuf.at[buf], out_slice(chunk), out_sems[buf]).start()

        @pl.when(chunk + 2 < _NCHK2)
        def _(buf=buf, chunk=chunk):
          pltpu.make_async_copy(
              in_slice(chunk + 2), xbuf.at[buf], in_sems[buf]).start()

      return carry

    lax.fori_loop(0, _NCHK2 // 2, pair, 0)

    pltpu.make_async_copy(
        obuf.at[0], out_slice(_NCHK2 - 2), out_sems[0]).wait()
    pltpu.make_async_copy(
        obuf.at[1], out_slice(_NCHK2 - 1), out_sems[1]).wait()

  return k(x4, s_sel, b_sel, t_sel)


def _expand(t):
  return jnp.broadcast_to(t[:, :, None], (QP_NUM, CHANNEL, 16)).reshape(
      QP_NUM, _XW)


@jax.jit
def kernel(x, index, h, b, a):
  idx = index.astype(jnp.int32)
  h2 = h.reshape(QP_NUM, CHANNEL)
  b2 = b.reshape(QP_NUM, CHANNEL)
  a2 = a.reshape(QP_NUM, CHANNEL)
  s_tab, t_tab = _tc_prep(h2, a2)
  s_sel, b_sel, t_sel = _sc_gather(
      _expand(s_tab), _expand(b2), _expand(t_tab), idx)
  xr = x.reshape(B, CHANNEL, _HF, 128)
  nb = B // 2
  o1 = _sc_elementwise(xr[:nb], s_sel, b_sel, t_sel, nb, 0)
  o2 = _sc_elementwise(xr[nb:], s_sel, b_sel, t_sel, nb, nb)
  return jnp.concatenate([o1, o2], axis=0).reshape(B, CHANNEL, H, W)


# speedup vs baseline: 1.2757x; 1.2757x over previous
import functools

import jax
import jax.numpy as jnp
from jax import lax
from jax.experimental import pallas as pl
from jax.experimental.pallas import tpu as pltpu
from jax.experimental.pallas import tpu_sc as plsc

QP_NUM = 64
CHANNEL = 192
B, H, W = 32, 64, 64
HW = H * W
_XW = CHANNEL * 16
_PERW = CHANNEL * HW
_CH = 4 * HW
_NCHK = _PERW // _CH
_UNROLL = 8

_GWORKERS = 4
_GPER = B // _GWORKERS


def _prep_body(h_ref, a_ref, s_ref, t_ref):
  s_ref[...] = jax.nn.softplus(h_ref[...])
  t_ref[...] = jnp.tanh(a_ref[...])


def _tc_prep(h2, a2):
  return pl.pallas_call(
      _prep_body,
      out_shape=[jax.ShapeDtypeStruct((QP_NUM, CHANNEL), jnp.float32)] * 2,
  )(h2, a2)


def _sc_gather(s_exp, b_exp, t_exp, idx):
  mesh = plsc.VectorSubcoreMesh(core_axis_name="c", subcore_axis_name="s")

  @functools.partial(
      pl.kernel,
      mesh=mesh,
      out_type=[jax.ShapeDtypeStruct((B, _XW), jnp.float32)] * 3,
      scratch_types=[
          pltpu.VMEM((_GPER,), jnp.int32),
          pltpu.VMEM((_GPER, _XW), jnp.float32),
          pltpu.VMEM((_GPER, _XW), jnp.float32),
          pltpu.VMEM((_GPER, _XW), jnp.float32),
          pltpu.SemaphoreType.DMA,
      ],
  )
  def k(s_hbm, b_hbm, t_hbm, idx_hbm, os, ob, ot, idx_v, rs, rb, rt, sem):
    wid = lax.axis_index("s") * 2 + lax.axis_index("c")

    @pl.when(wid < _GWORKERS)
    def _():
      base = wid * _GPER
      pltpu.sync_copy(idx_hbm.at[pl.ds(base, _GPER)], idx_v)
      pltpu.async_copy(s_hbm.at[idx_v], rs, sem).wait()
      pltpu.async_copy(b_hbm.at[idx_v], rb, sem).wait()
      pltpu.async_copy(t_hbm.at[idx_v], rt, sem).wait()
      pltpu.sync_copy(rs, os.at[pl.ds(base, _GPER)])
      pltpu.sync_copy(rb, ob.at[pl.ds(base, _GPER)])
      pltpu.sync_copy(rt, ot.at[pl.ds(base, _GPER)])

  return k(s_exp, b_exp, t_exp, idx)


_HF = HW // 128
_CROWS = 4
_NCHK2 = CHANNEL // _CROWS


def _sc_elementwise(x4, s_sel, b_sel, t_sel):
  mesh = plsc.VectorSubcoreMesh(core_axis_name="c", subcore_axis_name="s")

  @functools.partial(
      pl.kernel,
      mesh=mesh,
      out_type=jax.ShapeDtypeStruct((B, CHANNEL, _HF, 128), jnp.float32),
      scratch_types=[
          pltpu.VMEM((_XW,), jnp.float32),
          pltpu.VMEM((_XW,), jnp.float32),
          pltpu.VMEM((_XW,), jnp.float32),
          pltpu.VMEM((2, _CROWS, _HF, 128), jnp.float32),
          pltpu.VMEM((2, _CROWS, _HF, 128), jnp.float32),
          pltpu.SemaphoreType.DMA,
          pltpu.SemaphoreType.DMA,
          pltpu.SemaphoreType.DMA,
          pltpu.SemaphoreType.DMA,
      ],
  )
  def k(x_hbm, s_hbm, b_hbm, t_hbm, o_hbm,
        s_buf, b_buf, t_buf, xbuf, obuf,
        in_sem0, in_sem1, out_sem0, out_sem1):
    in_sems = (in_sem0, in_sem1)
    out_sems = (out_sem0, out_sem1)
    wid = lax.axis_index("s") * 2 + lax.axis_index("c")

    pltpu.sync_copy(s_hbm.at[wid], s_buf)
    pltpu.sync_copy(b_hbm.at[wid], b_buf)
    pltpu.sync_copy(t_hbm.at[wid], t_buf)

    def in_slice(chunk):
      return x_hbm.at[wid, pl.ds(chunk * _CROWS, _CROWS)]

    def out_slice(chunk):
      return o_hbm.at[wid, pl.ds(chunk * _CROWS, _CROWS)]

    pltpu.make_async_copy(in_slice(0), xbuf.at[0], in_sems[0]).start()
    pltpu.make_async_copy(in_slice(1), xbuf.at[1], in_sems[1]).start()

    def pair(g, carry):
      for buf in (0, 1):
        chunk = 2 * g + buf

        @pl.when(chunk >= 2)
        def _(buf=buf, chunk=chunk):
          pltpu.make_async_copy(
              obuf.at[buf], out_slice(chunk - 2), out_sems[buf]).wait()

        pltpu.make_async_copy(
            in_slice(chunk), xbuf.at[buf], in_sems[buf]).wait()

        xb = xbuf.at[buf]
        ob = obuf.at[buf]
        for r in range(_CROWS):
          c = chunk * _CROWS + r
          sv = s_buf[pl.ds(c * 16, 16)]
          bv = b_buf[pl.ds(c * 16, 16)]
          tv = t_buf[pl.ds(c * 16, 16)]
          t2v = tv + tv

          def inner(i, _, r=r, sv=sv, bv=bv, tv=tv, t2v=t2v, xb=xb, ob=ob):
            for u in range(128 // 16):
              v = xb[r, i, pl.ds(u * 16, 16)]
              y = v * sv + bv
              e = jnp.exp(y + y)
              q = t2v / (e + 1.0)
              ob[r, i, pl.ds(u * 16, 16)] = y + tv - q
            return 0

          lax.fori_loop(0, _HF, inner, 0)

        pltpu.make_async_copy(
            obuf.at[buf], out_slice(chunk), out_sems[buf]).start()

        @pl.when(chunk + 2 < _NCHK2)
        def _(buf=buf, chunk=chunk):
          pltpu.make_async_copy(
              in_slice(chunk + 2), xbuf.at[buf], in_sems[buf]).start()

      return carry

    lax.fori_loop(0, _NCHK2 // 2, pair, 0)

    pltpu.make_async_copy(
        obuf.at[0], out_slice(_NCHK2 - 2), out_sems[0]).wait()
    pltpu.make_async_copy(
        obuf.at[1], out_slice(_NCHK2 - 1), out_sems[1]).wait()

  return k(x4, s_sel, b_sel, t_sel)


def _expand(t):
  return jnp.broadcast_to(t[:, :, None], (QP_NUM, CHANNEL, 16)).reshape(
      QP_NUM, _XW)


@jax.jit
def kernel(x, index, h, b, a):
  idx = index.astype(jnp.int32)
  h2 = h.reshape(QP_NUM, CHANNEL)
  b2 = b.reshape(QP_NUM, CHANNEL)
  a2 = a.reshape(QP_NUM, CHANNEL)
  s_tab, t_tab = _tc_prep(h2, a2)
  s_sel, b_sel, t_sel = _sc_gather(
      _expand(s_tab), _expand(b2), _expand(t_tab), idx)
  out4 = _sc_elementwise(
      x.reshape(B, CHANNEL, _HF, 128), s_sel, b_sel, t_sel)
  return out4.reshape(B, CHANNEL, H, W)
